# Initial kernel scaffold; baseline (speedup 1.0000x reference)
import jax
import jax.numpy as jnp
from jax.experimental import pallas as pl


def kernel(w1r, b1, w2t, bn_scale, bn_shift, wfc1p, bfc1, wfc2t, bfc2, x):
    raise NotImplementedError("write your pallas kernel here")


# speedup vs baseline: 6.4882x; 6.4882x over previous
import jax
import jax.numpy as jnp
from jax.experimental import pallas as pl
from jax.experimental.pallas import tpu as pltpu

BT = 8
TM = 256


def _conv_stack_kernel(p1_ref, w1_ref, b1_ref, w2_ref, s2_ref, t2_ref,
                       o_ref, hp_ref):
    y1 = jnp.dot(p1_ref[...].reshape(BT * 832, 9), w1_ref[...],
                 preferred_element_type=jnp.float32)
    y1 = jnp.maximum(y1 + b1_ref[...], 0.0).astype(jnp.bfloat16)
    y1 = y1.reshape(BT, 26, 32, 32)

    acc = jnp.zeros((BT * 576, 64), jnp.float32)
    for kh in range(3):
        lhs = jnp.concatenate(
            [y1[:, kh:kh + 24, kw:kw + 24, :] for kw in range(3)],
            axis=-1).reshape(BT * 576, 96)
        acc = acc + jnp.dot(lhs, w2_ref[kh],
                            preferred_element_type=jnp.float32)

    z = jnp.maximum(acc * s2_ref[...] + t2_ref[...], 0.0)

    z4 = z.reshape(BT, 12, 2, 24, 64)
    hp_ref[...] = jnp.maximum(z4[:, :, 0], z4[:, :, 1]).reshape(BT * 288, 64)
    o_ref[...] = jnp.maximum(
        hp_ref[pl.ds(0, BT * 144, stride=2), :],
        hp_ref[pl.ds(1, BT * 144, stride=2), :],
    ).astype(jnp.bfloat16).reshape(BT, 144, 64)


def _conv_stack(p1, w1, b1, w2, s2, t2):
    B = p1.shape[0]
    return pl.pallas_call(
        _conv_stack_kernel,
        out_shape=jax.ShapeDtypeStruct((B, 144, 64), jnp.bfloat16),
        grid=(B // BT,),
        in_specs=[
            pl.BlockSpec((BT, 832, 9), lambda b: (b, 0, 0)),
            pl.BlockSpec((9, 32), lambda b: (0, 0)),
            pl.BlockSpec((1, 32), lambda b: (0, 0)),
            pl.BlockSpec((3, 96, 64), lambda b: (0, 0, 0)),
            pl.BlockSpec((1, 64), lambda b: (0, 0)),
            pl.BlockSpec((1, 64), lambda b: (0, 0)),
        ],
        out_specs=pl.BlockSpec((BT, 144, 64), lambda b: (b, 0, 0)),
        scratch_shapes=[pltpu.VMEM((BT * 288, 64), jnp.float32)],
        compiler_params=pltpu.CompilerParams(
            dimension_semantics=("parallel",),
            vmem_limit_bytes=64 * 1024 * 1024,
        ),
    )(p1, w1, b1, w2, s2, t2)


def _fc_head_kernel(x_ref, w1_ref, b1_ref, w2_ref, b2_ref, o_ref):
    h = jnp.dot(x_ref[...], w1_ref[...], preferred_element_type=jnp.float32)
    h = jnp.maximum(h + b1_ref[...], 0.0).astype(jnp.bfloat16)
    y = jnp.dot(h, w2_ref[...], preferred_element_type=jnp.float32)
    y = y + b2_ref[...]
    m = jnp.max(y, axis=-1, keepdims=True)
    lse = jnp.log(jnp.sum(jnp.exp(y - m), axis=-1, keepdims=True)) + m
    o_ref[...] = y - lse


def _fc_head(flat, w1, b1, w2, b2):
    B, K = flat.shape
    N1 = w1.shape[1]
    N2 = w2.shape[1]
    return pl.pallas_call(
        _fc_head_kernel,
        out_shape=jax.ShapeDtypeStruct((B, N2), jnp.float32),
        grid=(B // TM,),
        in_specs=[
            pl.BlockSpec((TM, K), lambda i: (i, 0)),
            pl.BlockSpec((K, N1), lambda i: (0, 0)),
            pl.BlockSpec((1, N1), lambda i: (0, 0)),
            pl.BlockSpec((N1, N2), lambda i: (0, 0)),
            pl.BlockSpec((1, N2), lambda i: (0, 0)),
        ],
        out_specs=pl.BlockSpec((TM, N2), lambda i: (i, 0)),
        compiler_params=pltpu.CompilerParams(
            dimension_semantics=("parallel",),
            vmem_limit_bytes=64 * 1024 * 1024,
        ),
    )(flat, w1, b1, w2, b2)


def kernel(w1r, b1, w2t, bn_scale, bn_shift, wfc1p, bfc1, wfc2t, bfc2, x):
    B = x.shape[0]
    xb = x.reshape(B, 28, 28).astype(jnp.bfloat16)

    taps = [xb[:, kh:kh + 26, kw:kw + 26] for kh in range(3) for kw in range(3)]
    p1 = jnp.stack(taps, axis=-1)
    p1 = jnp.pad(p1, ((0, 0), (0, 0), (0, 6), (0, 0))).reshape(B, 832, 9)

    w2g = w2t.reshape(3, 3 * 32, 64).astype(jnp.bfloat16)

    pooled = _conv_stack(p1, w1r.astype(jnp.bfloat16), b1, w2g,
                         bn_scale, bn_shift)
    flat = pooled.reshape(B, 9216)

    return _fc_head(flat, wfc1p.astype(jnp.bfloat16), bfc1,
                    wfc2t.astype(jnp.bfloat16), bfc2)
